# baseline (speedup 1.0000x reference)
import jax
import jax.numpy as jnp
from jax import lax
from jax.experimental import pallas as pl
from jax.experimental.pallas import tpu as pltpu
from jax.experimental.pallas import tpu_sc as plsc

_SCORE_THRESHOLD = 0.3
_IOU_THRESHOLD = 0.5
_TOP_K = 200
_MAX_NMS = 100
_B, _N, _C = 8, 5000, 21
_CP = 24
_NP = 5120
_NB = _NP // 16
_NEG = -1e30
_BIGI = 1 << 30


def _sc_nms_body(scores_hbm, boxes_hbm, out_hbm,
                 sw, bx, m1, oS, oY0, oX0, oY1, oX1, oM, shared):
    cid = lax.axis_index("c")
    sid = lax.axis_index("s")
    b = cid * 4 + sid // 4
    g = sid % 4

    for r in range(5):
        pltpu.sync_copy(scores_hbm.at[b, g, r], sw.at[pl.ds(r * _NP, _NP)])
    for r in range(4):
        pltpu.sync_copy(boxes_hbm.at[b, r], bx.at[pl.ds(r * _NP, _NP)])

    iota = lax.iota(jnp.int32, 16)
    zeros16 = jnp.zeros((16,), jnp.float32)
    negs16 = jnp.full((16,), _NEG, jnp.float32)
    lane0 = iota == 0

    def splat(v):
        return jnp.full((16,), v, jnp.int32)

    _TH = jnp.float32(_SCORE_THRESHOLD)

    def clear_lists(ci):
        co = ci * 128
        for j in range(8):
            li = co + j * 16 + iota
            for ref in (oS, oY0, oX0, oY1, oX1):
                plsc.store_scatter(ref, [li], zeros16)

    def build_m1(ci, m1b):
        cb = ci * _NP

        def m1_body(jv, _):
            acc = negs16
            for kk in range(16):
                idx = cb + jv * 256 + iota * 16 + kk
                acc = jnp.maximum(acc, plsc.load_gather(sw, [idx]))
            plsc.store_scatter(m1, [m1b + jv * 16 + iota], acc)
            return 0

        lax.fori_loop(0, _NB // 16, m1_body, 0)

    def build_m2(m1b):
        m2a = negs16
        for kk in range(16):
            m2a = jnp.maximum(m2a,
                              plsc.load_gather(m1, [m1b + iota * 16 + kk]))
        m2b = negs16
        for kk in range(16):
            idxm = m1b + jnp.minimum((16 + iota) * 16 + kk, _NB - 1)
            m2b = jnp.maximum(m2b, plsc.load_gather(m1, [idxm]))
        m2b = jnp.where(iota < 4, m2b, _NEG)
        return m2a, m2b

    def pop_once(st, ci, m1b):
        k, gm, v0, v1 = st
        cb = ci * _NP
        co = ci * 128
        alive = (k < _MAX_NMS) & (gm > _TH)
        c0 = jnp.where(v0 == gm, iota, _BIGI)
        c1 = jnp.where(v1 == gm, iota + 16, _BIGI)
        jstar = jnp.minimum(jnp.min(jnp.minimum(c0, c1)), 19)
        mv = plsc.load_gather(m1, [m1b + jstar * 16 + iota])
        bloc16 = jstar * 16 + iota
        bloc = jnp.minimum(jnp.min(jnp.where(mv == gm, bloc16, _BIGI)),
                           _NB - 1)
        si = bloc * 16 + iota
        sv = plsc.load_gather(sw, [cb + si])
        istar = jnp.minimum(jnp.min(jnp.where(sv == gm, si, _BIGI)), _NP - 1)
        ivec = splat(istar)
        by0 = plsc.load_gather(bx, [ivec])
        bx0 = plsc.load_gather(bx, [ivec + _NP])
        by1 = plsc.load_gather(bx, [ivec + 2 * _NP])
        bx1 = plsc.load_gather(bx, [ivec + 3 * _NP])
        a1 = jnp.maximum(by1 - by0, 0.0) * jnp.maximum(bx1 - bx0, 0.0)

        nk = jnp.where(alive, (k + 31) // 32, 0)

        def iou16(ki):
            ky0 = plsc.load_gather(oY0, [ki])
            kx0 = plsc.load_gather(oX0, [ki])
            ky1 = plsc.load_gather(oY1, [ki])
            kx1 = plsc.load_gather(oX1, [ki])
            iymin = jnp.maximum(by0, ky0)
            ixmin = jnp.maximum(bx0, kx0)
            iymax = jnp.minimum(by1, ky1)
            ixmax = jnp.minimum(bx1, kx1)
            inter = (jnp.maximum(iymax - iymin, 0.0) *
                     jnp.maximum(ixmax - ixmin, 0.0))
            a2 = (jnp.maximum(ky1 - ky0, 0.0) *
                  jnp.maximum(kx1 - kx0, 0.0))
            union = a1 + a2 - inter
            safe = jnp.where(union > 0, union, 1.0)
            return jnp.where(union > 0, inter / safe, 0.0)

        def iou_body(j, accmax):
            ki = co + j * 32 + iota
            return jnp.maximum(accmax,
                               jnp.maximum(iou16(ki), iou16(ki + 16)))

        accm = lax.fori_loop(0, nk, iou_body, zeros16)
        keep = (jnp.max(accm) <= _IOU_THRESHOLD) & alive
        kf = jnp.where(keep, 1.0, 0.0).astype(jnp.float32)
        wmask = jnp.logical_and(lane0, alive)

        plsc.store_scatter(sw, [ivec + cb], negs16, mask=wmask)
        nb = jnp.max(jnp.where(si == istar, _NEG, sv))
        plsc.store_scatter(m1, [splat(m1b + bloc)], jnp.full((16,), nb),
                           mask=wmask)
        nm2 = jnp.max(jnp.where(bloc16 == bloc, nb, mv))
        v0n = jnp.where(jnp.logical_and(iota == jstar, alive), nm2, v0)
        v1n = jnp.where(jnp.logical_and(iota + 16 == jstar, alive), nm2, v1)

        wl = splat(co + jnp.where(keep, k, 127))
        plsc.store_scatter(oS, [wl], jnp.full((16,), gm) * kf, mask=wmask)
        plsc.store_scatter(oY0, [wl], by0 * kf, mask=wmask)
        plsc.store_scatter(oX0, [wl], bx0 * kf, mask=wmask)
        plsc.store_scatter(oY1, [wl], by1 * kf, mask=wmask)
        plsc.store_scatter(oX1, [wl], bx1 * kf, mask=wmask)

        gm2 = jnp.max(jnp.maximum(v0n, v1n))
        return (k + keep.astype(jnp.int32), gm2, v0n, v1n)

    def start_state(m1b):
        v0, v1 = build_m2(m1b)
        return (jnp.int32(0), jnp.max(jnp.maximum(v0, v1)), v0, v1)

    def run_pair(ca, cbp):
        clear_lists(ca)
        clear_lists(cbp)
        build_m1(ca, 0)
        build_m1(cbp, _NB)

        def cond(st):
            (ka, ga, _a0, _a1), (kb, gb, _b0, _b1) = st
            return (((ka < _MAX_NMS) & (ga > _TH)) |
                    ((kb < _MAX_NMS) & (gb > _TH)))

        def body(st):
            sa, sb = st
            return (pop_once(sa, ca, 0), pop_once(sb, cbp, _NB))

        lax.while_loop(cond, body, (start_state(0), start_state(_NB)))

    def run_single(ci):
        clear_lists(ci)
        build_m1(ci, 0)

        def cond(st):
            k, gm, _v0, _v1 = st
            return (k < _MAX_NMS) & (gm > _TH)

        def body(st):
            return pop_once(st, ci, 0)

        lax.while_loop(cond, body, start_state(0))

    run_pair(0, 1)
    run_pair(2, 3)
    run_single(4)

    pltpu.sync_copy(oS, shared.at[sid, pl.ds(0, 640)])
    pltpu.sync_copy(oY0, shared.at[sid, pl.ds(640, 640)])
    pltpu.sync_copy(oX0, shared.at[sid, pl.ds(1280, 640)])
    pltpu.sync_copy(oY1, shared.at[sid, pl.ds(1920, 640)])
    pltpu.sync_copy(oX1, shared.at[sid, pl.ds(2560, 640)])
    plsc.subcore_barrier()

    @pl.when(sid % 4 == 0)
    def _():
        for j in range(4):
            pltpu.sync_copy(shared.at[sid + j], sw.at[pl.ds(j * 3200, 3200)])
        cc0 = iota
        cc1 = iota + 16
        base0 = (cc0 // 5) * 3200 + (cc0 % 5) * 128
        base1 = (cc1 // 5) * 3200 + (cc1 % 5) * 128

        def mstep(r, heads):
            h0, h1 = heads
            g0v = plsc.load_gather(sw, [base0 + jnp.minimum(h0, 127)])
            hs0 = jnp.where(h0 < 128, g0v, 0.0)
            g1v = plsc.load_gather(sw, [base1 + jnp.minimum(h1, 127)])
            hs1 = jnp.where((h1 < 128) & (cc1 < 20), g1v, _NEG)
            gmax = jnp.max(jnp.maximum(hs0, hs1))
            key0 = jnp.where(hs0 == gmax, cc0 * 256 + h0, _BIGI)
            key1 = jnp.where(hs1 == gmax, cc1 * 256 + h1, _BIGI)
            wkey = jnp.min(jnp.minimum(key0, key1))
            wcc = wkey // 256
            wh = wkey % 256
            basew = ((wcc // 5) * 3200 + (wcc % 5) * 128 +
                     jnp.minimum(wh, 127))
            cls_val = jnp.where(gmax > 0.25,
                                (wcc + 1).astype(jnp.float32), 0.0)
            plsc.store_scatter(oM, [splat(r)],
                               jnp.full((16,), cls_val), mask=lane0)
            plsc.store_scatter(oM, [splat(256 + r)],
                               jnp.full((16,), gmax), mask=lane0)
            for a in range(1, 5):
                va = plsc.load_gather(sw, [splat(basew + a * 640)])
                plsc.store_scatter(oM, [splat((a + 1) * 256 + r)], va,
                                   mask=lane0)
            h0n = h0 + (cc0 == wcc).astype(jnp.int32)
            h1n = h1 + (cc1 == wcc).astype(jnp.int32)
            return (h0n, h1n)

        zi = jnp.zeros((16,), jnp.int32)
        lax.fori_loop(0, _TOP_K, mstep, (zi, zi))
        for a in range(6):
            pltpu.sync_copy(oM.at[pl.ds(a * 256, 256)], out_hbm.at[b, a])


def _sc_nms(scores_t, boxes_t, interpret=False):
    return pl.kernel(
        _sc_nms_body,
        out_type=jax.ShapeDtypeStruct((_B, 6, 256), jnp.float32),
        mesh=plsc.VectorSubcoreMesh(core_axis_name="c", subcore_axis_name="s"),
        compiler_params=pltpu.CompilerParams(use_tc_tiling_on_sc=False,
                                             needs_layout_passes=False),
        scratch_types=[
            pltpu.VMEM((5 * _NP,), jnp.float32),
            pltpu.VMEM((4 * _NP,), jnp.float32),
            pltpu.VMEM((2 * _NB,), jnp.float32),
            pltpu.VMEM((640,), jnp.float32),
            pltpu.VMEM((640,), jnp.float32),
            pltpu.VMEM((640,), jnp.float32),
            pltpu.VMEM((640,), jnp.float32),
            pltpu.VMEM((640,), jnp.float32),
            pltpu.VMEM((1536,), jnp.float32),
            pltpu.VMEM_SHARED((16, 3200), jnp.float32),
        ],
        interpret=interpret,
    )(scores_t, boxes_t)


def _merge_body(sS_ref, sY0_ref, sX0_ref, sY1_ref, sX1_ref, out_ref, merged):
    lane128 = lax.broadcasted_iota(jnp.int32, (_B, _CP, 128), 2)
    row_iota = lax.broadcasted_iota(jnp.int32, (_B, _CP, 1), 1)
    sS = sS_ref[...]
    cls_e = jnp.where(sS > 0.25, (row_iota + 1).astype(jnp.float32), 0.0)
    sY0 = sY0_ref[...]
    sX0 = sX0_ref[...]
    sY1 = sY1_ref[...]
    sX1 = sX1_ref[...]

    row8 = lax.broadcasted_iota(jnp.int32, (_B, 8, 256), 1)
    lane256 = lax.broadcasted_iota(jnp.int32, (_B, 8, 256), 2)
    merged[...] = jnp.zeros((_B, 8, 256), jnp.float32)

    def mstep(r, heads):
        hoh = lane128 == heads
        hs = jnp.sum(jnp.where(hoh, sS, 0.0), axis=2, keepdims=True)
        best = jnp.max(hs, axis=1, keepdims=True)
        flat = row_iota * _MAX_NMS + heads
        wflat = jnp.min(jnp.where(hs == best, flat, _BIGI), axis=1,
                        keepdims=True)
        wrow = flat == wflat
        woh = (wrow & hoh).astype(jnp.float32)
        vals = [jnp.sum(jnp.sum(woh * a, axis=2, keepdims=True), axis=1,
                        keepdims=True)
                for a in (cls_e, sS, sY0, sX0, sY1, sX1)]
        col = jnp.zeros((_B, 8, 256), jnp.float32)
        for k, v in enumerate(vals):
            col = col + jnp.where(row8 == k, v, 0.0)
        merged[...] = jnp.where(lane256 == r, col, merged[...])
        return heads + wrow.astype(jnp.int32)

    lax.fori_loop(0, _TOP_K, mstep, jnp.zeros((_B, _CP, 1), jnp.int32))
    out_ref[...] = merged[...]


def _merge(sS, sY0, sX0, sY1, sX1, interpret=False):
    return pl.pallas_call(
        _merge_body,
        out_shape=jax.ShapeDtypeStruct((_B, 8, 256), jnp.float32),
        scratch_shapes=[pltpu.VMEM((_B, 8, 256), jnp.float32)],
        interpret=interpret,
    )(sS, sY0, sX0, sY1, sX1)


def kernel(scores_pred, boxes_pred, _interpret=False):
    scores_t = jnp.transpose(scores_pred[:, :, 1:], (0, 2, 1))
    scores_t = jnp.pad(scores_t, ((0, 0), (0, 0), (0, _NP - _N)))
    scores_t = scores_t.reshape(_B, 4, 5, _NP)
    boxes_t = jnp.transpose(boxes_pred, (0, 2, 1))
    boxes_t = jnp.pad(boxes_t, ((0, 0), (0, 0), (0, _NP - _N)))
    res = _sc_nms(scores_t, boxes_t, interpret=_interpret)
    cls = res[:, 0, :_TOP_K]
    score = res[:, 1, :_TOP_K]
    top_scores = jnp.stack([cls, score], axis=-1)
    top_boxes = jnp.transpose(res[:, 2:6, :_TOP_K], (0, 2, 1))
    return top_scores, top_boxes


# speedup vs baseline: 1.0727x; 1.0727x over previous
import jax
import jax.numpy as jnp
from jax import lax
from jax.experimental import pallas as pl
from jax.experimental.pallas import tpu as pltpu
from jax.experimental.pallas import tpu_sc as plsc

_SCORE_THRESHOLD = 0.3
_IOU_THRESHOLD = 0.5
_TOP_K = 200
_MAX_NMS = 100
_B, _N, _C = 8, 5000, 21
_CP = 24
_NP = 5120
_NB = _NP // 16
_NEG = -1e30
_BIGI = 1 << 30


def _sc_nms_body(scores_hbm, boxes_hbm, out_hbm,
                 sw, bx, m1, oS, oY0, oX0, oY1, oX1, oM, shared):
    cid = lax.axis_index("c")
    sid = lax.axis_index("s")
    b = cid * 4 + sid // 4
    g = sid % 4

    for r in range(5):
        pltpu.sync_copy(scores_hbm.at[b, g, r], sw.at[pl.ds(r * _NP, _NP)])
    for r in range(4):
        pltpu.sync_copy(boxes_hbm.at[b, r], bx.at[pl.ds(r * _NP, _NP)])

    iota = lax.iota(jnp.int32, 16)
    zeros16 = jnp.zeros((16,), jnp.float32)
    negs16 = jnp.full((16,), _NEG, jnp.float32)
    lane0 = iota == 0

    def splat(v):
        return jnp.full((16,), v, jnp.int32)

    _TH = jnp.float32(_SCORE_THRESHOLD)

    def clear_lists(ci):
        co = ci * 128
        for j in range(8):
            li = co + j * 16 + iota
            for ref in (oS, oY0, oX0, oY1, oX1):
                plsc.store_scatter(ref, [li], zeros16)

    def build_m1(ci, m1b):
        cb = ci * _NP

        def m1_body(jv, _):
            acc = negs16
            for kk in range(16):
                idx = cb + jv * 256 + iota * 16 + kk
                acc = jnp.maximum(acc, plsc.load_gather(sw, [idx]))
            plsc.store_scatter(m1, [m1b + jv * 16 + iota], acc)
            return 0

        lax.fori_loop(0, _NB // 16, m1_body, 0)

    def build_m2(m1b):
        m2a = negs16
        for kk in range(16):
            m2a = jnp.maximum(m2a,
                              plsc.load_gather(m1, [m1b + iota * 16 + kk]))
        m2b = negs16
        for kk in range(16):
            idxm = m1b + jnp.minimum((16 + iota) * 16 + kk, _NB - 1)
            m2b = jnp.maximum(m2b, plsc.load_gather(m1, [idxm]))
        m2b = jnp.where(iota < 4, m2b, _NEG)
        return m2a, m2b

    def select_phase(st, ci, m1b):
        k, gm, v0, v1 = st
        cb = ci * _NP
        alive = (k < _MAX_NMS) & (gm > _TH)
        c0 = jnp.where(v0 == gm, iota, _BIGI)
        c1 = jnp.where(v1 == gm, iota + 16, _BIGI)
        jstar = jnp.minimum(jnp.min(jnp.minimum(c0, c1)), 19)
        mv = plsc.load_gather(m1, [m1b + jstar * 16 + iota])
        bloc16 = jstar * 16 + iota
        bloc = jnp.minimum(jnp.min(jnp.where(mv == gm, bloc16, _BIGI)),
                           _NB - 1)
        si = bloc * 16 + iota
        sv = plsc.load_gather(sw, [cb + si])
        istar = jnp.minimum(jnp.min(jnp.where(sv == gm, si, _BIGI)), _NP - 1)
        ivec = splat(istar)
        by0 = plsc.load_gather(bx, [ivec])
        bx0 = plsc.load_gather(bx, [ivec + _NP])
        by1 = plsc.load_gather(bx, [ivec + 2 * _NP])
        bx1 = plsc.load_gather(bx, [ivec + 3 * _NP])
        a1 = jnp.maximum(by1 - by0, 0.0) * jnp.maximum(bx1 - bx0, 0.0)
        nk = jnp.where(alive, (k + 31) // 32, 0)
        return (alive, jstar, mv, bloc16, bloc, si, sv, istar, ivec,
                by0, bx0, by1, bx1, a1, nk)

    def iou16(sel, ki):
        (_al, _js, _mv, _b16, _bl, _si, _sv, _is, _iv,
         by0, bx0, by1, bx1, a1, _nk) = sel
        ky0 = plsc.load_gather(oY0, [ki])
        kx0 = plsc.load_gather(oX0, [ki])
        ky1 = plsc.load_gather(oY1, [ki])
        kx1 = plsc.load_gather(oX1, [ki])
        iymin = jnp.maximum(by0, ky0)
        ixmin = jnp.maximum(bx0, kx0)
        iymax = jnp.minimum(by1, ky1)
        ixmax = jnp.minimum(bx1, kx1)
        inter = (jnp.maximum(iymax - iymin, 0.0) *
                 jnp.maximum(ixmax - ixmin, 0.0))
        a2 = (jnp.maximum(ky1 - ky0, 0.0) *
              jnp.maximum(kx1 - kx0, 0.0))
        union = a1 + a2 - inter
        safe = jnp.where(union > 0, union, 1.0)
        return jnp.where(union > 0, inter / safe, 0.0)

    def iou_pass(sel, ci):
        co = ci * 128

        def iou_body(j, accmax):
            ki = co + j * 32 + iota
            return jnp.maximum(accmax, jnp.maximum(iou16(sel, ki),
                                                   iou16(sel, ki + 16)))

        return lax.fori_loop(0, sel[-1], iou_body, zeros16)

    def finish_phase(st, sel, accm, ci, m1b):
        k, gm, v0, v1 = st
        cb = ci * _NP
        co = ci * 128
        (alive, jstar, mv, bloc16, bloc, si, sv, istar, ivec,
         by0, bx0, by1, bx1, a1, _nk) = sel
        keep = (jnp.max(accm) <= _IOU_THRESHOLD) & alive
        kf = jnp.where(keep, 1.0, 0.0).astype(jnp.float32)
        wmask = jnp.logical_and(lane0, alive)

        plsc.store_scatter(sw, [ivec + cb], negs16, mask=wmask)
        nb = jnp.max(jnp.where(si == istar, _NEG, sv))
        plsc.store_scatter(m1, [splat(m1b + bloc)], jnp.full((16,), nb),
                           mask=wmask)
        nm2 = jnp.max(jnp.where(bloc16 == bloc, nb, mv))
        v0n = jnp.where(jnp.logical_and(iota == jstar, alive), nm2, v0)
        v1n = jnp.where(jnp.logical_and(iota + 16 == jstar, alive), nm2, v1)

        wl = splat(co + jnp.where(keep, k, 127))
        plsc.store_scatter(oS, [wl], jnp.full((16,), gm) * kf, mask=wmask)
        plsc.store_scatter(oY0, [wl], by0 * kf, mask=wmask)
        plsc.store_scatter(oX0, [wl], bx0 * kf, mask=wmask)
        plsc.store_scatter(oY1, [wl], by1 * kf, mask=wmask)
        plsc.store_scatter(oX1, [wl], bx1 * kf, mask=wmask)

        gm2 = jnp.max(jnp.maximum(v0n, v1n))
        return (k + keep.astype(jnp.int32), gm2, v0n, v1n)

    def pop_once(st, ci, m1b):
        sel = select_phase(st, ci, m1b)
        accm = iou_pass(sel, ci)
        return finish_phase(st, sel, accm, ci, m1b)

    def start_state(m1b):
        v0, v1 = build_m2(m1b)
        return (jnp.int32(0), jnp.max(jnp.maximum(v0, v1)), v0, v1)

    def run_pair(ca, cbp):
        clear_lists(ca)
        clear_lists(cbp)
        build_m1(ca, 0)
        build_m1(cbp, _NB)

        def cond(st):
            (ka, ga, _a0, _a1), (kb, gb, _b0, _b1) = st
            return (((ka < _MAX_NMS) & (ga > _TH)) |
                    ((kb < _MAX_NMS) & (gb > _TH)))

        def body(st):
            sa, sb = st
            sela = select_phase(sa, ca, 0)
            selb = select_phase(sb, cbp, _NB)
            coa = ca * 128
            cob = cbp * 128
            nk = jnp.maximum(sela[-1], selb[-1])

            def iou_body(j, accs):
                accA, accB = accs
                kia = coa + j * 32 + iota
                kib = cob + j * 32 + iota
                accA = jnp.maximum(accA, jnp.maximum(iou16(sela, kia),
                                                     iou16(sela, kia + 16)))
                accB = jnp.maximum(accB, jnp.maximum(iou16(selb, kib),
                                                     iou16(selb, kib + 16)))
                return (accA, accB)

            accA, accB = lax.fori_loop(0, nk, iou_body, (zeros16, zeros16))
            return (finish_phase(sa, sela, accA, ca, 0),
                    finish_phase(sb, selb, accB, cbp, _NB))

        lax.while_loop(cond, body, (start_state(0), start_state(_NB)))

    def run_single(ci):
        clear_lists(ci)
        build_m1(ci, 0)

        def cond(st):
            k, gm, _v0, _v1 = st
            return (k < _MAX_NMS) & (gm > _TH)

        def body(st):
            return pop_once(st, ci, 0)

        lax.while_loop(cond, body, start_state(0))

    run_pair(0, 1)
    run_pair(2, 3)
    run_single(4)

    pltpu.sync_copy(oS, shared.at[sid, pl.ds(0, 640)])
    pltpu.sync_copy(oY0, shared.at[sid, pl.ds(640, 640)])
    pltpu.sync_copy(oX0, shared.at[sid, pl.ds(1280, 640)])
    pltpu.sync_copy(oY1, shared.at[sid, pl.ds(1920, 640)])
    pltpu.sync_copy(oX1, shared.at[sid, pl.ds(2560, 640)])
    plsc.subcore_barrier()

    @pl.when(sid % 4 == 0)
    def _():
        for j in range(4):
            pltpu.sync_copy(shared.at[sid + j], sw.at[pl.ds(j * 3200, 3200)])
        cc0 = iota
        cc1 = iota + 16
        base0 = (cc0 // 5) * 3200 + (cc0 % 5) * 128
        base1 = (cc1 // 5) * 3200 + (cc1 % 5) * 128

        def mstep(r, heads):
            h0, h1 = heads
            g0v = plsc.load_gather(sw, [base0 + jnp.minimum(h0, 127)])
            hs0 = jnp.where(h0 < 128, g0v, 0.0)
            g1v = plsc.load_gather(sw, [base1 + jnp.minimum(h1, 127)])
            hs1 = jnp.where((h1 < 128) & (cc1 < 20), g1v, _NEG)
            gmax = jnp.max(jnp.maximum(hs0, hs1))
            key0 = jnp.where(hs0 == gmax, cc0 * 256 + h0, _BIGI)
            key1 = jnp.where(hs1 == gmax, cc1 * 256 + h1, _BIGI)
            wkey = jnp.min(jnp.minimum(key0, key1))
            wcc = wkey // 256
            wh = wkey % 256
            basew = ((wcc // 5) * 3200 + (wcc % 5) * 128 +
                     jnp.minimum(wh, 127))
            cls_val = jnp.where(gmax > 0.25,
                                (wcc + 1).astype(jnp.float32), 0.0)
            plsc.store_scatter(oM, [splat(r)],
                               jnp.full((16,), cls_val), mask=lane0)
            plsc.store_scatter(oM, [splat(256 + r)],
                               jnp.full((16,), gmax), mask=lane0)
            for a in range(1, 5):
                va = plsc.load_gather(sw, [splat(basew + a * 640)])
                plsc.store_scatter(oM, [splat((a + 1) * 256 + r)], va,
                                   mask=lane0)
            h0n = h0 + (cc0 == wcc).astype(jnp.int32)
            h1n = h1 + (cc1 == wcc).astype(jnp.int32)
            return (h0n, h1n)

        zi = jnp.zeros((16,), jnp.int32)
        lax.fori_loop(0, _TOP_K, mstep, (zi, zi))
        for a in range(6):
            pltpu.sync_copy(oM.at[pl.ds(a * 256, 256)], out_hbm.at[b, a])


def _sc_nms(scores_t, boxes_t, interpret=False):
    return pl.kernel(
        _sc_nms_body,
        out_type=jax.ShapeDtypeStruct((_B, 6, 256), jnp.float32),
        mesh=plsc.VectorSubcoreMesh(core_axis_name="c", subcore_axis_name="s"),
        compiler_params=pltpu.CompilerParams(use_tc_tiling_on_sc=False,
                                             needs_layout_passes=False),
        scratch_types=[
            pltpu.VMEM((5 * _NP,), jnp.float32),
            pltpu.VMEM((4 * _NP,), jnp.float32),
            pltpu.VMEM((2 * _NB,), jnp.float32),
            pltpu.VMEM((640,), jnp.float32),
            pltpu.VMEM((640,), jnp.float32),
            pltpu.VMEM((640,), jnp.float32),
            pltpu.VMEM((640,), jnp.float32),
            pltpu.VMEM((640,), jnp.float32),
            pltpu.VMEM((1536,), jnp.float32),
            pltpu.VMEM_SHARED((16, 3200), jnp.float32),
        ],
        interpret=interpret,
    )(scores_t, boxes_t)


def _merge_body(sS_ref, sY0_ref, sX0_ref, sY1_ref, sX1_ref, out_ref, merged):
    lane128 = lax.broadcasted_iota(jnp.int32, (_B, _CP, 128), 2)
    row_iota = lax.broadcasted_iota(jnp.int32, (_B, _CP, 1), 1)
    sS = sS_ref[...]
    cls_e = jnp.where(sS > 0.25, (row_iota + 1).astype(jnp.float32), 0.0)
    sY0 = sY0_ref[...]
    sX0 = sX0_ref[...]
    sY1 = sY1_ref[...]
    sX1 = sX1_ref[...]

    row8 = lax.broadcasted_iota(jnp.int32, (_B, 8, 256), 1)
    lane256 = lax.broadcasted_iota(jnp.int32, (_B, 8, 256), 2)
    merged[...] = jnp.zeros((_B, 8, 256), jnp.float32)

    def mstep(r, heads):
        hoh = lane128 == heads
        hs = jnp.sum(jnp.where(hoh, sS, 0.0), axis=2, keepdims=True)
        best = jnp.max(hs, axis=1, keepdims=True)
        flat = row_iota * _MAX_NMS + heads
        wflat = jnp.min(jnp.where(hs == best, flat, _BIGI), axis=1,
                        keepdims=True)
        wrow = flat == wflat
        woh = (wrow & hoh).astype(jnp.float32)
        vals = [jnp.sum(jnp.sum(woh * a, axis=2, keepdims=True), axis=1,
                        keepdims=True)
                for a in (cls_e, sS, sY0, sX0, sY1, sX1)]
        col = jnp.zeros((_B, 8, 256), jnp.float32)
        for k, v in enumerate(vals):
            col = col + jnp.where(row8 == k, v, 0.0)
        merged[...] = jnp.where(lane256 == r, col, merged[...])
        return heads + wrow.astype(jnp.int32)

    lax.fori_loop(0, _TOP_K, mstep, jnp.zeros((_B, _CP, 1), jnp.int32))
    out_ref[...] = merged[...]


def _merge(sS, sY0, sX0, sY1, sX1, interpret=False):
    return pl.pallas_call(
        _merge_body,
        out_shape=jax.ShapeDtypeStruct((_B, 8, 256), jnp.float32),
        scratch_shapes=[pltpu.VMEM((_B, 8, 256), jnp.float32)],
        interpret=interpret,
    )(sS, sY0, sX0, sY1, sX1)


def kernel(scores_pred, boxes_pred, _interpret=False):
    scores_t = jnp.transpose(scores_pred[:, :, 1:], (0, 2, 1))
    scores_t = jnp.pad(scores_t, ((0, 0), (0, 0), (0, _NP - _N)))
    scores_t = scores_t.reshape(_B, 4, 5, _NP)
    boxes_t = jnp.transpose(boxes_pred, (0, 2, 1))
    boxes_t = jnp.pad(boxes_t, ((0, 0), (0, 0), (0, _NP - _N)))
    res = _sc_nms(scores_t, boxes_t, interpret=_interpret)
    cls = res[:, 0, :_TOP_K]
    score = res[:, 1, :_TOP_K]
    top_scores = jnp.stack([cls, score], axis=-1)
    top_boxes = jnp.transpose(res[:, 2:6, :_TOP_K], (0, 2, 1))
    return top_scores, top_boxes


# speedup vs baseline: 1.1356x; 1.0586x over previous
import jax
import jax.numpy as jnp
from jax import lax
from jax.experimental import pallas as pl
from jax.experimental.pallas import tpu as pltpu
from jax.experimental.pallas import tpu_sc as plsc

_SCORE_THRESHOLD = 0.3
_IOU_THRESHOLD = 0.5
_TOP_K = 200
_MAX_NMS = 100
_B, _N, _C = 8, 5000, 21
_CP = 24
_NP = 5120
_NB = _NP // 16
_NEG = -1e30
_BIGI = 1 << 30


def _sc_nms_body(scores_hbm, boxes_hbm, out_hbm,
                 sw, bx, m1, oS, oY0, oX0, oY1, oX1, oM, shared):
    cid = lax.axis_index("c")
    sid = lax.axis_index("s")
    b = cid * 4 + sid // 4
    g = sid % 4

    for r in range(5):
        pltpu.sync_copy(scores_hbm.at[b, g, r], sw.at[pl.ds(r * _NP, _NP)])
    for r in range(4):
        pltpu.sync_copy(boxes_hbm.at[b, r], bx.at[pl.ds(r * _NP, _NP)])

    iota = lax.iota(jnp.int32, 16)
    zeros16 = jnp.zeros((16,), jnp.float32)
    negs16 = jnp.full((16,), _NEG, jnp.float32)
    lane0 = iota == 0

    def splat(v):
        return jnp.full((16,), v, jnp.int32)

    _TH = jnp.float32(_SCORE_THRESHOLD)

    def clear_lists(ci):
        co = ci * 128
        for j in range(8):
            li = co + j * 16 + iota
            for ref in (oS, oY0, oX0, oY1, oX1):
                plsc.store_scatter(ref, [li], zeros16)

    def build_m1(ci, m1b):
        cb = ci * _NP

        def m1_body(jv, _):
            acc = negs16
            for kk in range(16):
                idx = cb + jv * 256 + iota * 16 + kk
                acc = jnp.maximum(acc, plsc.load_gather(sw, [idx]))
            plsc.store_scatter(m1, [m1b + jv * 16 + iota], acc)
            return 0

        lax.fori_loop(0, _NB // 16, m1_body, 0)

    def build_m2(m1b):
        m2a = negs16
        for kk in range(16):
            m2a = jnp.maximum(m2a,
                              plsc.load_gather(m1, [m1b + iota * 16 + kk]))
        m2b = negs16
        for kk in range(16):
            idxm = m1b + jnp.minimum((16 + iota) * 16 + kk, _NB - 1)
            m2b = jnp.maximum(m2b, plsc.load_gather(m1, [idxm]))
        m2b = jnp.where(iota < 4, m2b, _NEG)
        return m2a, m2b

    def select_phase(st, ci, m1b):
        k, gm, v0, v1 = st
        cb = ci * _NP
        alive = (k < _MAX_NMS) & (gm > _TH)
        c0 = jnp.where(v0 == gm, iota, _BIGI)
        c1 = jnp.where(v1 == gm, iota + 16, _BIGI)
        jstar = jnp.minimum(jnp.min(jnp.minimum(c0, c1)), 19)
        mv = plsc.load_gather(m1, [m1b + jstar * 16 + iota])
        bloc16 = jstar * 16 + iota
        bloc = jnp.minimum(jnp.min(jnp.where(mv == gm, bloc16, _BIGI)),
                           _NB - 1)
        si = bloc * 16 + iota
        sv = plsc.load_gather(sw, [cb + si])
        istar = jnp.minimum(jnp.min(jnp.where(sv == gm, si, _BIGI)), _NP - 1)
        ivec = splat(istar)
        by0 = plsc.load_gather(bx, [ivec])
        bx0 = plsc.load_gather(bx, [ivec + _NP])
        by1 = plsc.load_gather(bx, [ivec + 2 * _NP])
        bx1 = plsc.load_gather(bx, [ivec + 3 * _NP])
        a1 = jnp.maximum(by1 - by0, 0.0) * jnp.maximum(bx1 - bx0, 0.0)
        nk = jnp.where(alive, (k + 31) // 32, 0)
        return (alive, jstar, mv, bloc16, bloc, si, sv, istar, ivec,
                by0, bx0, by1, bx1, a1, nk)

    def iou16(sel, ki):
        (_al, _js, _mv, _b16, _bl, _si, _sv, _is, _iv,
         by0, bx0, by1, bx1, a1, _nk) = sel
        ky0 = plsc.load_gather(oY0, [ki])
        kx0 = plsc.load_gather(oX0, [ki])
        ky1 = plsc.load_gather(oY1, [ki])
        kx1 = plsc.load_gather(oX1, [ki])
        iymin = jnp.maximum(by0, ky0)
        ixmin = jnp.maximum(bx0, kx0)
        iymax = jnp.minimum(by1, ky1)
        ixmax = jnp.minimum(bx1, kx1)
        inter = (jnp.maximum(iymax - iymin, 0.0) *
                 jnp.maximum(ixmax - ixmin, 0.0))
        a2 = (jnp.maximum(ky1 - ky0, 0.0) *
              jnp.maximum(kx1 - kx0, 0.0))
        union = a1 + a2 - inter
        safe = jnp.where(union > 0, union, 1.0)
        return jnp.where(union > 0, inter / safe, 0.0)

    def iou_pass(sel, ci):
        co = ci * 128

        def iou_body(j, accmax):
            ki = co + j * 32 + iota
            return jnp.maximum(accmax, jnp.maximum(iou16(sel, ki),
                                                   iou16(sel, ki + 16)))

        return lax.fori_loop(0, sel[-1], iou_body, zeros16)

    def finish_phase(st, sel, accm, ci, m1b):
        k, gm, v0, v1 = st
        cb = ci * _NP
        co = ci * 128
        (alive, jstar, mv, bloc16, bloc, si, sv, istar, ivec,
         by0, bx0, by1, bx1, a1, _nk) = sel
        keep = (jnp.max(accm) <= _IOU_THRESHOLD) & alive
        kf = jnp.where(keep, 1.0, 0.0).astype(jnp.float32)
        wmask = jnp.logical_and(lane0, alive)

        plsc.store_scatter(sw, [ivec + cb], negs16, mask=wmask)
        nb = jnp.max(jnp.where(si == istar, _NEG, sv))
        plsc.store_scatter(m1, [splat(m1b + bloc)], jnp.full((16,), nb),
                           mask=wmask)
        nm2 = jnp.max(jnp.where(bloc16 == bloc, nb, mv))
        v0n = jnp.where(jnp.logical_and(iota == jstar, alive), nm2, v0)
        v1n = jnp.where(jnp.logical_and(iota + 16 == jstar, alive), nm2, v1)

        wl = splat(co + jnp.where(keep, k, 127))
        plsc.store_scatter(oS, [wl], jnp.full((16,), gm) * kf, mask=wmask)
        plsc.store_scatter(oY0, [wl], by0 * kf, mask=wmask)
        plsc.store_scatter(oX0, [wl], bx0 * kf, mask=wmask)
        plsc.store_scatter(oY1, [wl], by1 * kf, mask=wmask)
        plsc.store_scatter(oX1, [wl], bx1 * kf, mask=wmask)

        gm2 = jnp.max(jnp.maximum(v0n, v1n))
        return (k + keep.astype(jnp.int32), gm2, v0n, v1n)

    def pop_once(st, ci, m1b):
        sel = select_phase(st, ci, m1b)
        accm = iou_pass(sel, ci)
        return finish_phase(st, sel, accm, ci, m1b)

    def start_state(m1b):
        v0, v1 = build_m2(m1b)
        return (jnp.int32(0), jnp.max(jnp.maximum(v0, v1)), v0, v1)

    def run_multi(cis):
        for idx, ci in enumerate(cis):
            clear_lists(ci)
            build_m1(ci, idx * _NB)

        def cond(st):
            alive = [(k < _MAX_NMS) & (gm > _TH) for (k, gm, _v0, _v1) in st]
            r = alive[0]
            for a in alive[1:]:
                r = r | a
            return r

        def body(st):
            sels = [select_phase(s, ci, idx * _NB)
                    for idx, (s, ci) in enumerate(zip(st, cis))]
            nk = sels[0][-1]
            for sel in sels[1:]:
                nk = jnp.maximum(nk, sel[-1])

            def iou_body(j, accs):
                out = []
                for sel, ci, acc in zip(sels, cis, accs):
                    ki = ci * 128 + j * 32 + iota
                    out.append(jnp.maximum(acc,
                                           jnp.maximum(iou16(sel, ki),
                                                       iou16(sel, ki + 16))))
                return tuple(out)

            accs = lax.fori_loop(0, nk, iou_body,
                                 tuple(zeros16 for _ in cis))
            return tuple(finish_phase(s, sel, acc, ci, idx * _NB)
                         for idx, (s, sel, acc, ci)
                         in enumerate(zip(st, sels, accs, cis)))

        lax.while_loop(cond, body,
                       tuple(start_state(idx * _NB)
                             for idx in range(len(cis))))

    run_multi((0, 1, 2))
    run_multi((3, 4))

    pltpu.sync_copy(oS, shared.at[sid, pl.ds(0, 640)])
    pltpu.sync_copy(oY0, shared.at[sid, pl.ds(640, 640)])
    pltpu.sync_copy(oX0, shared.at[sid, pl.ds(1280, 640)])
    pltpu.sync_copy(oY1, shared.at[sid, pl.ds(1920, 640)])
    pltpu.sync_copy(oX1, shared.at[sid, pl.ds(2560, 640)])
    plsc.subcore_barrier()

    @pl.when(sid % 4 == 0)
    def _():
        for j in range(4):
            pltpu.sync_copy(shared.at[sid + j], sw.at[pl.ds(j * 3200, 3200)])
        cc0 = iota
        cc1 = iota + 16
        base0 = (cc0 // 5) * 3200 + (cc0 % 5) * 128
        base1 = (cc1 // 5) * 3200 + (cc1 % 5) * 128

        def mstep(r, heads):
            h0, h1 = heads
            g0v = plsc.load_gather(sw, [base0 + jnp.minimum(h0, 127)])
            hs0 = jnp.where(h0 < 128, g0v, 0.0)
            g1v = plsc.load_gather(sw, [base1 + jnp.minimum(h1, 127)])
            hs1 = jnp.where((h1 < 128) & (cc1 < 20), g1v, _NEG)
            gmax = jnp.max(jnp.maximum(hs0, hs1))
            key0 = jnp.where(hs0 == gmax, cc0 * 256 + h0, _BIGI)
            key1 = jnp.where(hs1 == gmax, cc1 * 256 + h1, _BIGI)
            wkey = jnp.min(jnp.minimum(key0, key1))
            wcc = wkey // 256
            wh = wkey % 256
            basew = ((wcc // 5) * 3200 + (wcc % 5) * 128 +
                     jnp.minimum(wh, 127))
            cls_val = jnp.where(gmax > 0.25,
                                (wcc + 1).astype(jnp.float32), 0.0)
            plsc.store_scatter(oM, [splat(r)],
                               jnp.full((16,), cls_val), mask=lane0)
            plsc.store_scatter(oM, [splat(256 + r)],
                               jnp.full((16,), gmax), mask=lane0)
            for a in range(1, 5):
                va = plsc.load_gather(sw, [splat(basew + a * 640)])
                plsc.store_scatter(oM, [splat((a + 1) * 256 + r)], va,
                                   mask=lane0)
            h0n = h0 + (cc0 == wcc).astype(jnp.int32)
            h1n = h1 + (cc1 == wcc).astype(jnp.int32)
            return (h0n, h1n)

        zi = jnp.zeros((16,), jnp.int32)
        lax.fori_loop(0, _TOP_K, mstep, (zi, zi))
        for a in range(6):
            pltpu.sync_copy(oM.at[pl.ds(a * 256, 256)], out_hbm.at[b, a])


def _sc_nms(scores_t, boxes_t, interpret=False):
    return pl.kernel(
        _sc_nms_body,
        out_type=jax.ShapeDtypeStruct((_B, 6, 256), jnp.float32),
        mesh=plsc.VectorSubcoreMesh(core_axis_name="c", subcore_axis_name="s"),
        compiler_params=pltpu.CompilerParams(use_tc_tiling_on_sc=False,
                                             needs_layout_passes=False),
        scratch_types=[
            pltpu.VMEM((5 * _NP,), jnp.float32),
            pltpu.VMEM((4 * _NP,), jnp.float32),
            pltpu.VMEM((3 * _NB,), jnp.float32),
            pltpu.VMEM((640,), jnp.float32),
            pltpu.VMEM((640,), jnp.float32),
            pltpu.VMEM((640,), jnp.float32),
            pltpu.VMEM((640,), jnp.float32),
            pltpu.VMEM((640,), jnp.float32),
            pltpu.VMEM((1536,), jnp.float32),
            pltpu.VMEM_SHARED((16, 3200), jnp.float32),
        ],
        interpret=interpret,
    )(scores_t, boxes_t)


def _merge_body(sS_ref, sY0_ref, sX0_ref, sY1_ref, sX1_ref, out_ref, merged):
    lane128 = lax.broadcasted_iota(jnp.int32, (_B, _CP, 128), 2)
    row_iota = lax.broadcasted_iota(jnp.int32, (_B, _CP, 1), 1)
    sS = sS_ref[...]
    cls_e = jnp.where(sS > 0.25, (row_iota + 1).astype(jnp.float32), 0.0)
    sY0 = sY0_ref[...]
    sX0 = sX0_ref[...]
    sY1 = sY1_ref[...]
    sX1 = sX1_ref[...]

    row8 = lax.broadcasted_iota(jnp.int32, (_B, 8, 256), 1)
    lane256 = lax.broadcasted_iota(jnp.int32, (_B, 8, 256), 2)
    merged[...] = jnp.zeros((_B, 8, 256), jnp.float32)

    def mstep(r, heads):
        hoh = lane128 == heads
        hs = jnp.sum(jnp.where(hoh, sS, 0.0), axis=2, keepdims=True)
        best = jnp.max(hs, axis=1, keepdims=True)
        flat = row_iota * _MAX_NMS + heads
        wflat = jnp.min(jnp.where(hs == best, flat, _BIGI), axis=1,
                        keepdims=True)
        wrow = flat == wflat
        woh = (wrow & hoh).astype(jnp.float32)
        vals = [jnp.sum(jnp.sum(woh * a, axis=2, keepdims=True), axis=1,
                        keepdims=True)
                for a in (cls_e, sS, sY0, sX0, sY1, sX1)]
        col = jnp.zeros((_B, 8, 256), jnp.float32)
        for k, v in enumerate(vals):
            col = col + jnp.where(row8 == k, v, 0.0)
        merged[...] = jnp.where(lane256 == r, col, merged[...])
        return heads + wrow.astype(jnp.int32)

    lax.fori_loop(0, _TOP_K, mstep, jnp.zeros((_B, _CP, 1), jnp.int32))
    out_ref[...] = merged[...]


def _merge(sS, sY0, sX0, sY1, sX1, interpret=False):
    return pl.pallas_call(
        _merge_body,
        out_shape=jax.ShapeDtypeStruct((_B, 8, 256), jnp.float32),
        scratch_shapes=[pltpu.VMEM((_B, 8, 256), jnp.float32)],
        interpret=interpret,
    )(sS, sY0, sX0, sY1, sX1)


def kernel(scores_pred, boxes_pred, _interpret=False):
    scores_t = jnp.transpose(scores_pred[:, :, 1:], (0, 2, 1))
    scores_t = jnp.pad(scores_t, ((0, 0), (0, 0), (0, _NP - _N)))
    scores_t = scores_t.reshape(_B, 4, 5, _NP)
    boxes_t = jnp.transpose(boxes_pred, (0, 2, 1))
    boxes_t = jnp.pad(boxes_t, ((0, 0), (0, 0), (0, _NP - _N)))
    res = _sc_nms(scores_t, boxes_t, interpret=_interpret)
    cls = res[:, 0, :_TOP_K]
    score = res[:, 1, :_TOP_K]
    top_scores = jnp.stack([cls, score], axis=-1)
    top_boxes = jnp.transpose(res[:, 2:6, :_TOP_K], (0, 2, 1))
    return top_scores, top_boxes


# speedup vs baseline: 1.1486x; 1.0115x over previous
import jax
import jax.numpy as jnp
from jax import lax
from jax.experimental import pallas as pl
from jax.experimental.pallas import tpu as pltpu
from jax.experimental.pallas import tpu_sc as plsc

_SCORE_THRESHOLD = 0.3
_IOU_THRESHOLD = 0.5
_TOP_K = 200
_MAX_NMS = 100
_B, _N, _C = 8, 5000, 21
_CP = 24
_NP = 5120
_NB = _NP // 16
_NEG = -1e30
_BIGI = 1 << 30


def _sc_nms_body(scores_hbm, boxes_hbm, out_hbm,
                 sw, bx, m1, oS, oY0, oX0, oY1, oX1, oM, shared):
    cid = lax.axis_index("c")
    sid = lax.axis_index("s")
    b = cid * 4 + sid // 4
    g = sid % 4

    for r in range(5):
        pltpu.sync_copy(scores_hbm.at[b, g, r], sw.at[pl.ds(r * _NP, _NP)])
    for r in range(4):
        pltpu.sync_copy(boxes_hbm.at[b, r], bx.at[pl.ds(r * _NP, _NP)])

    iota = lax.iota(jnp.int32, 16)
    zeros16 = jnp.zeros((16,), jnp.float32)
    negs16 = jnp.full((16,), _NEG, jnp.float32)
    lane0 = iota == 0

    def splat(v):
        return jnp.full((16,), v, jnp.int32)

    _TH = jnp.float32(_SCORE_THRESHOLD)

    def clear_lists(ci):
        co = ci * 128
        for j in range(8):
            li = co + j * 16 + iota
            for ref in (oS, oY0, oX0, oY1, oX1):
                plsc.store_scatter(ref, [li], zeros16)

    def build_m1(ci, m1b):
        cb = ci * _NP

        def m1_body(jv, _):
            acc = negs16
            for kk in range(16):
                idx = cb + jv * 256 + iota * 16 + kk
                acc = jnp.maximum(acc, plsc.load_gather(sw, [idx]))
            plsc.store_scatter(m1, [m1b + jv * 16 + iota], acc)
            return 0

        lax.fori_loop(0, _NB // 16, m1_body, 0)

    def build_m2(m1b):
        m2a = negs16
        for kk in range(16):
            m2a = jnp.maximum(m2a,
                              plsc.load_gather(m1, [m1b + iota * 16 + kk]))
        m2b = negs16
        for kk in range(16):
            idxm = m1b + jnp.minimum((16 + iota) * 16 + kk, _NB - 1)
            m2b = jnp.maximum(m2b, plsc.load_gather(m1, [idxm]))
        m2b = jnp.where(iota < 4, m2b, _NEG)
        return m2a, m2b

    def select_phase(st, ci, m1b):
        k, gm, v0, v1 = st
        cb = ci * _NP
        alive = (k < _MAX_NMS) & (gm > _TH)
        c0 = jnp.where(v0 == gm, iota, _BIGI)
        c1 = jnp.where(v1 == gm, iota + 16, _BIGI)
        jstar = jnp.minimum(jnp.min(jnp.minimum(c0, c1)), 19)
        mv = plsc.load_gather(m1, [m1b + jstar * 16 + iota])
        bloc16 = jstar * 16 + iota
        bloc = jnp.minimum(jnp.min(jnp.where(mv == gm, bloc16, _BIGI)),
                           _NB - 1)
        si = bloc * 16 + iota
        sv = plsc.load_gather(sw, [cb + si])
        istar = jnp.minimum(jnp.min(jnp.where(sv == gm, si, _BIGI)), _NP - 1)
        ivec = splat(istar)
        by0 = plsc.load_gather(bx, [ivec])
        bx0 = plsc.load_gather(bx, [ivec + _NP])
        by1 = plsc.load_gather(bx, [ivec + 2 * _NP])
        bx1 = plsc.load_gather(bx, [ivec + 3 * _NP])
        a1 = jnp.maximum(by1 - by0, 0.0) * jnp.maximum(bx1 - bx0, 0.0)
        nk = jnp.where(alive, (k + 31) // 32, 0)
        return (alive, jstar, mv, bloc16, bloc, si, sv, istar, ivec,
                by0, bx0, by1, bx1, a1, nk)

    def iou16(sel, ki):
        (_al, _js, _mv, _b16, _bl, _si, _sv, _is, _iv,
         by0, bx0, by1, bx1, a1, _nk) = sel
        ky0 = plsc.load_gather(oY0, [ki])
        kx0 = plsc.load_gather(oX0, [ki])
        ky1 = plsc.load_gather(oY1, [ki])
        kx1 = plsc.load_gather(oX1, [ki])
        iymin = jnp.maximum(by0, ky0)
        ixmin = jnp.maximum(bx0, kx0)
        iymax = jnp.minimum(by1, ky1)
        ixmax = jnp.minimum(bx1, kx1)
        inter = (jnp.maximum(iymax - iymin, 0.0) *
                 jnp.maximum(ixmax - ixmin, 0.0))
        a2 = (jnp.maximum(ky1 - ky0, 0.0) *
              jnp.maximum(kx1 - kx0, 0.0))
        union = a1 + a2 - inter
        safe = jnp.where(union > 0, union, 1.0)
        return jnp.where(union > 0, inter / safe, 0.0)

    def iou_pass(sel, ci):
        co = ci * 128

        def iou_body(j, accmax):
            ki = co + j * 32 + iota
            return jnp.maximum(accmax, jnp.maximum(iou16(sel, ki),
                                                   iou16(sel, ki + 16)))

        return lax.fori_loop(0, sel[-1], iou_body, zeros16)

    def finish_phase(st, sel, accm, ci, m1b):
        k, gm, v0, v1 = st
        cb = ci * _NP
        co = ci * 128
        (alive, jstar, mv, bloc16, bloc, si, sv, istar, ivec,
         by0, bx0, by1, bx1, a1, _nk) = sel
        keep = (jnp.max(accm) <= _IOU_THRESHOLD) & alive
        kf = jnp.where(keep, 1.0, 0.0).astype(jnp.float32)
        wmask = jnp.logical_and(lane0, alive)

        plsc.store_scatter(sw, [ivec + cb], negs16, mask=wmask)
        nb = jnp.max(jnp.where(si == istar, _NEG, sv))
        plsc.store_scatter(m1, [splat(m1b + bloc)], jnp.full((16,), nb),
                           mask=wmask)
        nm2 = jnp.max(jnp.where(bloc16 == bloc, nb, mv))
        v0n = jnp.where(jnp.logical_and(iota == jstar, alive), nm2, v0)
        v1n = jnp.where(jnp.logical_and(iota + 16 == jstar, alive), nm2, v1)

        wl = splat(co + jnp.where(keep, k, 127))
        plsc.store_scatter(oS, [wl], jnp.full((16,), gm) * kf, mask=wmask)
        plsc.store_scatter(oY0, [wl], by0 * kf, mask=wmask)
        plsc.store_scatter(oX0, [wl], bx0 * kf, mask=wmask)
        plsc.store_scatter(oY1, [wl], by1 * kf, mask=wmask)
        plsc.store_scatter(oX1, [wl], bx1 * kf, mask=wmask)

        gm2 = jnp.max(jnp.maximum(v0n, v1n))
        return (k + keep.astype(jnp.int32), gm2, v0n, v1n)

    def pop_once(st, ci, m1b):
        sel = select_phase(st, ci, m1b)
        accm = iou_pass(sel, ci)
        return finish_phase(st, sel, accm, ci, m1b)

    def start_state(m1b):
        v0, v1 = build_m2(m1b)
        return (jnp.int32(0), jnp.max(jnp.maximum(v0, v1)), v0, v1)

    def run_multi(cis):
        for idx, ci in enumerate(cis):
            clear_lists(ci)
            build_m1(ci, idx * _NB)

        def cond(st):
            alive = [(k < _MAX_NMS) & (gm > _TH) for (k, gm, _v0, _v1) in st]
            r = alive[0]
            for a in alive[1:]:
                r = r | a
            return r

        def body(st):
            sels = [select_phase(s, ci, idx * _NB)
                    for idx, (s, ci) in enumerate(zip(st, cis))]
            nk = sels[0][-1]
            for sel in sels[1:]:
                nk = jnp.maximum(nk, sel[-1])

            def iou_body(j, accs):
                out = []
                for sel, ci, acc in zip(sels, cis, accs):
                    ki = ci * 128 + j * 32 + iota
                    out.append(jnp.maximum(acc,
                                           jnp.maximum(iou16(sel, ki),
                                                       iou16(sel, ki + 16))))
                return tuple(out)

            accs = lax.fori_loop(0, nk, iou_body,
                                 tuple(zeros16 for _ in cis))
            return tuple(finish_phase(s, sel, acc, ci, idx * _NB)
                         for idx, (s, sel, acc, ci)
                         in enumerate(zip(st, sels, accs, cis)))

        lax.while_loop(cond, body,
                       tuple(start_state(idx * _NB)
                             for idx in range(len(cis))))

    run_multi((0, 1, 2, 3, 4))

    pltpu.sync_copy(oS, shared.at[sid, pl.ds(0, 640)])
    pltpu.sync_copy(oY0, shared.at[sid, pl.ds(640, 640)])
    pltpu.sync_copy(oX0, shared.at[sid, pl.ds(1280, 640)])
    pltpu.sync_copy(oY1, shared.at[sid, pl.ds(1920, 640)])
    pltpu.sync_copy(oX1, shared.at[sid, pl.ds(2560, 640)])
    plsc.subcore_barrier()

    @pl.when(sid % 4 == 0)
    def _():
        for j in range(4):
            pltpu.sync_copy(shared.at[sid + j], sw.at[pl.ds(j * 3200, 3200)])
        cc0 = iota
        cc1 = iota + 16
        base0 = (cc0 // 5) * 3200 + (cc0 % 5) * 128
        base1 = (cc1 // 5) * 3200 + (cc1 % 5) * 128

        def mstep(r, heads):
            h0, h1 = heads
            g0v = plsc.load_gather(sw, [base0 + jnp.minimum(h0, 127)])
            hs0 = jnp.where(h0 < 128, g0v, 0.0)
            g1v = plsc.load_gather(sw, [base1 + jnp.minimum(h1, 127)])
            hs1 = jnp.where((h1 < 128) & (cc1 < 20), g1v, _NEG)
            gmax = jnp.max(jnp.maximum(hs0, hs1))
            key0 = jnp.where(hs0 == gmax, cc0 * 256 + h0, _BIGI)
            key1 = jnp.where(hs1 == gmax, cc1 * 256 + h1, _BIGI)
            wkey = jnp.min(jnp.minimum(key0, key1))
            wcc = wkey // 256
            wh = wkey % 256
            basew = ((wcc // 5) * 3200 + (wcc % 5) * 128 +
                     jnp.minimum(wh, 127))
            cls_val = jnp.where(gmax > 0.25,
                                (wcc + 1).astype(jnp.float32), 0.0)
            plsc.store_scatter(oM, [splat(r)],
                               jnp.full((16,), cls_val), mask=lane0)
            plsc.store_scatter(oM, [splat(256 + r)],
                               jnp.full((16,), gmax), mask=lane0)
            for a in range(1, 5):
                va = plsc.load_gather(sw, [splat(basew + a * 640)])
                plsc.store_scatter(oM, [splat((a + 1) * 256 + r)], va,
                                   mask=lane0)
            h0n = h0 + (cc0 == wcc).astype(jnp.int32)
            h1n = h1 + (cc1 == wcc).astype(jnp.int32)
            return (h0n, h1n)

        zi = jnp.zeros((16,), jnp.int32)
        lax.fori_loop(0, _TOP_K, mstep, (zi, zi))
        for a in range(6):
            pltpu.sync_copy(oM.at[pl.ds(a * 256, 256)], out_hbm.at[b, a])


def _sc_nms(scores_t, boxes_t, interpret=False):
    return pl.kernel(
        _sc_nms_body,
        out_type=jax.ShapeDtypeStruct((_B, 6, 256), jnp.float32),
        mesh=plsc.VectorSubcoreMesh(core_axis_name="c", subcore_axis_name="s"),
        compiler_params=pltpu.CompilerParams(use_tc_tiling_on_sc=False,
                                             needs_layout_passes=False),
        scratch_types=[
            pltpu.VMEM((5 * _NP,), jnp.float32),
            pltpu.VMEM((4 * _NP,), jnp.float32),
            pltpu.VMEM((5 * _NB,), jnp.float32),
            pltpu.VMEM((640,), jnp.float32),
            pltpu.VMEM((640,), jnp.float32),
            pltpu.VMEM((640,), jnp.float32),
            pltpu.VMEM((640,), jnp.float32),
            pltpu.VMEM((640,), jnp.float32),
            pltpu.VMEM((1536,), jnp.float32),
            pltpu.VMEM_SHARED((16, 3200), jnp.float32),
        ],
        interpret=interpret,
    )(scores_t, boxes_t)


def _merge_body(sS_ref, sY0_ref, sX0_ref, sY1_ref, sX1_ref, out_ref, merged):
    lane128 = lax.broadcasted_iota(jnp.int32, (_B, _CP, 128), 2)
    row_iota = lax.broadcasted_iota(jnp.int32, (_B, _CP, 1), 1)
    sS = sS_ref[...]
    cls_e = jnp.where(sS > 0.25, (row_iota + 1).astype(jnp.float32), 0.0)
    sY0 = sY0_ref[...]
    sX0 = sX0_ref[...]
    sY1 = sY1_ref[...]
    sX1 = sX1_ref[...]

    row8 = lax.broadcasted_iota(jnp.int32, (_B, 8, 256), 1)
    lane256 = lax.broadcasted_iota(jnp.int32, (_B, 8, 256), 2)
    merged[...] = jnp.zeros((_B, 8, 256), jnp.float32)

    def mstep(r, heads):
        hoh = lane128 == heads
        hs = jnp.sum(jnp.where(hoh, sS, 0.0), axis=2, keepdims=True)
        best = jnp.max(hs, axis=1, keepdims=True)
        flat = row_iota * _MAX_NMS + heads
        wflat = jnp.min(jnp.where(hs == best, flat, _BIGI), axis=1,
                        keepdims=True)
        wrow = flat == wflat
        woh = (wrow & hoh).astype(jnp.float32)
        vals = [jnp.sum(jnp.sum(woh * a, axis=2, keepdims=True), axis=1,
                        keepdims=True)
                for a in (cls_e, sS, sY0, sX0, sY1, sX1)]
        col = jnp.zeros((_B, 8, 256), jnp.float32)
        for k, v in enumerate(vals):
            col = col + jnp.where(row8 == k, v, 0.0)
        merged[...] = jnp.where(lane256 == r, col, merged[...])
        return heads + wrow.astype(jnp.int32)

    lax.fori_loop(0, _TOP_K, mstep, jnp.zeros((_B, _CP, 1), jnp.int32))
    out_ref[...] = merged[...]


def _merge(sS, sY0, sX0, sY1, sX1, interpret=False):
    return pl.pallas_call(
        _merge_body,
        out_shape=jax.ShapeDtypeStruct((_B, 8, 256), jnp.float32),
        scratch_shapes=[pltpu.VMEM((_B, 8, 256), jnp.float32)],
        interpret=interpret,
    )(sS, sY0, sX0, sY1, sX1)


def kernel(scores_pred, boxes_pred, _interpret=False):
    scores_t = jnp.transpose(scores_pred[:, :, 1:], (0, 2, 1))
    scores_t = jnp.pad(scores_t, ((0, 0), (0, 0), (0, _NP - _N)))
    scores_t = scores_t.reshape(_B, 4, 5, _NP)
    boxes_t = jnp.transpose(boxes_pred, (0, 2, 1))
    boxes_t = jnp.pad(boxes_t, ((0, 0), (0, 0), (0, _NP - _N)))
    res = _sc_nms(scores_t, boxes_t, interpret=_interpret)
    cls = res[:, 0, :_TOP_K]
    score = res[:, 1, :_TOP_K]
    top_scores = jnp.stack([cls, score], axis=-1)
    top_boxes = jnp.transpose(res[:, 2:6, :_TOP_K], (0, 2, 1))
    return top_scores, top_boxes


# speedup vs baseline: 1.1496x; 1.0008x over previous
import jax
import jax.numpy as jnp
from jax import lax
from jax.experimental import pallas as pl
from jax.experimental.pallas import tpu as pltpu
from jax.experimental.pallas import tpu_sc as plsc

_SCORE_THRESHOLD = 0.3
_IOU_THRESHOLD = 0.5
_TOP_K = 200
_MAX_NMS = 100
_B, _N, _C = 8, 5000, 21
_CP = 24
_NP = 5120
_NB = _NP // 16
_NEG = -1e30
_BIGI = 1 << 30


def _sc_nms_body(scores_hbm, boxes_hbm, out_hbm,
                 sw, bx, m1, oS, oY0, oX0, oY1, oX1, oM, shared):
    cid = lax.axis_index("c")
    sid = lax.axis_index("s")
    b = cid * 4 + sid // 4
    g = sid % 4

    for r in range(5):
        pltpu.sync_copy(scores_hbm.at[b, g, r], sw.at[pl.ds(r * _NP, _NP)])
    for r in range(4):
        pltpu.sync_copy(boxes_hbm.at[b, r], bx.at[pl.ds(r * _NP, _NP)])

    iota = lax.iota(jnp.int32, 16)
    zeros16 = jnp.zeros((16,), jnp.float32)
    negs16 = jnp.full((16,), _NEG, jnp.float32)
    lane0 = iota == 0

    def splat(v):
        return jnp.full((16,), v, jnp.int32)

    _TH = jnp.float32(_SCORE_THRESHOLD)

    def clear_lists(ci):
        co = ci * 128
        for j in range(8):
            li = co + j * 16 + iota
            for ref in (oS, oY0, oX0, oY1, oX1):
                plsc.store_scatter(ref, [li], zeros16)

    def build_m1(ci, m1b):
        cb = ci * _NP

        def m1_body(jv, _):
            acc = negs16
            for kk in range(16):
                idx = cb + jv * 256 + iota * 16 + kk
                acc = jnp.maximum(acc, plsc.load_gather(sw, [idx]))
            plsc.store_scatter(m1, [m1b + jv * 16 + iota], acc)
            return 0

        lax.fori_loop(0, _NB // 16, m1_body, 0)

    def build_m2(m1b):
        m2a = negs16
        for kk in range(16):
            m2a = jnp.maximum(m2a,
                              plsc.load_gather(m1, [m1b + iota * 16 + kk]))
        m2b = negs16
        for kk in range(16):
            idxm = m1b + jnp.minimum((16 + iota) * 16 + kk, _NB - 1)
            m2b = jnp.maximum(m2b, plsc.load_gather(m1, [idxm]))
        m2b = jnp.where(iota < 4, m2b, _NEG)
        return m2a, m2b

    def select_phase(st, ci, m1b):
        k, gm, v0, v1 = st
        cb = ci * _NP
        alive = (k < _MAX_NMS) & (gm > _TH)
        c0 = jnp.where(v0 == gm, iota, _BIGI)
        c1 = jnp.where(v1 == gm, iota + 16, _BIGI)
        jstar = jnp.minimum(jnp.min(jnp.minimum(c0, c1)), 19)
        mv = plsc.load_gather(m1, [m1b + jstar * 16 + iota])
        bloc16 = jstar * 16 + iota
        bloc = jnp.minimum(jnp.min(jnp.where(mv == gm, bloc16, _BIGI)),
                           _NB - 1)
        si = bloc * 16 + iota
        sv = plsc.load_gather(sw, [cb + si])
        istar = jnp.minimum(jnp.min(jnp.where(sv == gm, si, _BIGI)), _NP - 1)
        ivec = splat(istar)
        by0 = plsc.load_gather(bx, [ivec])
        bx0 = plsc.load_gather(bx, [ivec + _NP])
        by1 = plsc.load_gather(bx, [ivec + 2 * _NP])
        bx1 = plsc.load_gather(bx, [ivec + 3 * _NP])
        a1 = jnp.maximum(by1 - by0, 0.0) * jnp.maximum(bx1 - bx0, 0.0)
        nk = jnp.where(alive, (k + 31) // 32, 0)
        return (alive, jstar, mv, bloc16, bloc, si, sv, istar, ivec,
                by0, bx0, by1, bx1, a1, nk)

    def iou16(sel, ki):
        (_al, _js, _mv, _b16, _bl, _si, _sv, _is, _iv,
         by0, bx0, by1, bx1, a1, _nk) = sel
        ky0 = plsc.load_gather(oY0, [ki])
        kx0 = plsc.load_gather(oX0, [ki])
        ky1 = plsc.load_gather(oY1, [ki])
        kx1 = plsc.load_gather(oX1, [ki])
        iymin = jnp.maximum(by0, ky0)
        ixmin = jnp.maximum(bx0, kx0)
        iymax = jnp.minimum(by1, ky1)
        ixmax = jnp.minimum(bx1, kx1)
        inter = (jnp.maximum(iymax - iymin, 0.0) *
                 jnp.maximum(ixmax - ixmin, 0.0))
        a2 = (jnp.maximum(ky1 - ky0, 0.0) *
              jnp.maximum(kx1 - kx0, 0.0))
        union = a1 + a2 - inter
        safe = jnp.where(union > 0, union, 1.0)
        return jnp.where(union > 0, inter / safe, 0.0)

    def iou_pass(sel, ci):
        co = ci * 128

        def iou_body(j, accmax):
            ki = co + j * 32 + iota
            return jnp.maximum(accmax, jnp.maximum(iou16(sel, ki),
                                                   iou16(sel, ki + 16)))

        return lax.fori_loop(0, sel[-1], iou_body, zeros16)

    def finish_phase(st, sel, accm, ci, m1b):
        k, gm, v0, v1 = st
        cb = ci * _NP
        co = ci * 128
        (alive, jstar, mv, bloc16, bloc, si, sv, istar, ivec,
         by0, bx0, by1, bx1, a1, _nk) = sel
        keep = (jnp.max(accm) <= _IOU_THRESHOLD) & alive
        kf = jnp.where(keep, 1.0, 0.0).astype(jnp.float32)
        wmask = jnp.logical_and(lane0, alive)

        plsc.store_scatter(sw, [ivec + cb], negs16, mask=wmask)
        nb = jnp.max(jnp.where(si == istar, _NEG, sv))
        plsc.store_scatter(m1, [splat(m1b + bloc)], jnp.full((16,), nb),
                           mask=wmask)
        nm2 = jnp.max(jnp.where(bloc16 == bloc, nb, mv))
        v0n = jnp.where(jnp.logical_and(iota == jstar, alive), nm2, v0)
        v1n = jnp.where(jnp.logical_and(iota + 16 == jstar, alive), nm2, v1)

        wl = splat(co + jnp.where(keep, k, 127))
        plsc.store_scatter(oS, [wl], jnp.full((16,), gm) * kf, mask=wmask)
        plsc.store_scatter(oY0, [wl], by0 * kf, mask=wmask)
        plsc.store_scatter(oX0, [wl], bx0 * kf, mask=wmask)
        plsc.store_scatter(oY1, [wl], by1 * kf, mask=wmask)
        plsc.store_scatter(oX1, [wl], bx1 * kf, mask=wmask)

        gm2 = jnp.max(jnp.maximum(v0n, v1n))
        return (k + keep.astype(jnp.int32), gm2, v0n, v1n)

    def pop_once(st, ci, m1b):
        sel = select_phase(st, ci, m1b)
        accm = iou_pass(sel, ci)
        return finish_phase(st, sel, accm, ci, m1b)

    def start_state(m1b):
        v0, v1 = build_m2(m1b)
        return (jnp.int32(0), jnp.max(jnp.maximum(v0, v1)), v0, v1)

    def run_multi(cis):
        for idx, ci in enumerate(cis):
            clear_lists(ci)
            build_m1(ci, idx * _NB)

        def cond(st):
            alive = [(k < _MAX_NMS) & (gm > _TH) for (k, gm, _v0, _v1) in st]
            r = alive[0]
            for a in alive[1:]:
                r = r | a
            return r

        def body(st):
            sels = [select_phase(s, ci, idx * _NB)
                    for idx, (s, ci) in enumerate(zip(st, cis))]
            nk = sels[0][-1]
            for sel in sels[1:]:
                nk = jnp.maximum(nk, sel[-1])

            def iou_body(j, accs):
                out = []
                for sel, ci, acc in zip(sels, cis, accs):
                    ki = ci * 128 + j * 32 + iota
                    out.append(jnp.maximum(acc,
                                           jnp.maximum(iou16(sel, ki),
                                                       iou16(sel, ki + 16))))
                return tuple(out)

            accs = lax.fori_loop(0, nk, iou_body,
                                 tuple(zeros16 for _ in cis))
            return tuple(finish_phase(s, sel, acc, ci, idx * _NB)
                         for idx, (s, sel, acc, ci)
                         in enumerate(zip(st, sels, accs, cis)))

        lax.while_loop(cond, body,
                       tuple(start_state(idx * _NB)
                             for idx in range(len(cis))))

    run_multi((0, 1, 2, 3, 4))

    pltpu.sync_copy(oS, shared.at[sid, pl.ds(0, 640)])
    pltpu.sync_copy(oY0, shared.at[sid, pl.ds(640, 640)])
    pltpu.sync_copy(oX0, shared.at[sid, pl.ds(1280, 640)])
    pltpu.sync_copy(oY1, shared.at[sid, pl.ds(1920, 640)])
    pltpu.sync_copy(oX1, shared.at[sid, pl.ds(2560, 640)])
    plsc.subcore_barrier()

    @pl.when(sid % 4 == 0)
    def _():
        for j in range(4):
            pltpu.sync_copy(shared.at[sid + j], sw.at[pl.ds(j * 3200, 3200)])
        cc0 = iota
        cc1 = iota + 16
        base0 = (cc0 // 5) * 3200 + (cc0 % 5) * 128
        base1 = (cc1 // 5) * 3200 + (cc1 % 5) * 128

        def mstep(r, heads):
            h0, h1 = heads
            g0v = plsc.load_gather(sw, [base0 + jnp.minimum(h0, 127)])
            hs0 = jnp.where(h0 < 128, g0v, 0.0)
            g1v = plsc.load_gather(sw, [base1 + jnp.minimum(h1, 127)])
            hs1 = jnp.where((h1 < 128) & (cc1 < 20), g1v, _NEG)
            gmax = jnp.max(jnp.maximum(hs0, hs1))
            key0 = jnp.where(hs0 == gmax, cc0 * 256 + h0, _BIGI)
            key1 = jnp.where(hs1 == gmax, cc1 * 256 + h1, _BIGI)
            wkey = jnp.min(jnp.minimum(key0, key1))
            wcc = wkey // 256
            wh = wkey % 256
            basew = ((wcc // 5) * 3200 + (wcc % 5) * 128 +
                     jnp.minimum(wh, 127))
            cls_val = jnp.where(gmax > 0.25,
                                (wcc + 1).astype(jnp.float32), 0.0)
            plsc.store_scatter(oM, [splat(r)],
                               jnp.full((16,), cls_val), mask=lane0)
            plsc.store_scatter(oM, [splat(256 + r)],
                               jnp.full((16,), gmax), mask=lane0)
            for a in range(1, 5):
                va = plsc.load_gather(sw, [splat(basew + a * 640)])
                plsc.store_scatter(oM, [splat((a + 1) * 256 + r)], va,
                                   mask=lane0)
            h0n = h0 + (cc0 == wcc).astype(jnp.int32)
            h1n = h1 + (cc1 == wcc).astype(jnp.int32)
            return (h0n, h1n)

        zi = jnp.zeros((16,), jnp.int32)
        lax.fori_loop(0, _TOP_K, mstep, (zi, zi))
        for a in range(6):
            pltpu.sync_copy(oM.at[pl.ds(a * 256, 256)], out_hbm.at[b, a])


def _sc_nms(scores_t, boxes_t, interpret=False):
    return pl.kernel(
        _sc_nms_body,
        out_type=jax.ShapeDtypeStruct((_B, 6, 256), jnp.float32),
        mesh=plsc.VectorSubcoreMesh(core_axis_name="c", subcore_axis_name="s"),
        compiler_params=pltpu.CompilerParams(use_tc_tiling_on_sc=False,
                                             needs_layout_passes=False),
        scratch_types=[
            pltpu.VMEM((5 * _NP,), jnp.float32),
            pltpu.VMEM((4 * _NP,), jnp.float32),
            pltpu.VMEM((5 * _NB,), jnp.float32),
            pltpu.VMEM((640,), jnp.float32),
            pltpu.VMEM((640,), jnp.float32),
            pltpu.VMEM((640,), jnp.float32),
            pltpu.VMEM((640,), jnp.float32),
            pltpu.VMEM((640,), jnp.float32),
            pltpu.VMEM((1536,), jnp.float32),
            pltpu.VMEM_SHARED((16, 3200), jnp.float32),
        ],
        interpret=interpret,
    )(scores_t, boxes_t)


def kernel(scores_pred, boxes_pred, _interpret=False):
    scores_t = jnp.transpose(scores_pred[:, :, 1:], (0, 2, 1))
    scores_t = jnp.pad(scores_t, ((0, 0), (0, 0), (0, _NP - _N)))
    scores_t = scores_t.reshape(_B, 4, 5, _NP)
    boxes_t = jnp.transpose(boxes_pred, (0, 2, 1))
    boxes_t = jnp.pad(boxes_t, ((0, 0), (0, 0), (0, _NP - _N)))
    res = _sc_nms(scores_t, boxes_t, interpret=_interpret)
    cls = res[:, 0, :_TOP_K]
    score = res[:, 1, :_TOP_K]
    top_scores = jnp.stack([cls, score], axis=-1)
    top_boxes = jnp.transpose(res[:, 2:6, :_TOP_K], (0, 2, 1))
    return top_scores, top_boxes
